# baseline (speedup 1.0000x reference)
import jax
import jax.numpy as jnp
from jax import lax
from jax.experimental import pallas as pl
from jax.experimental.pallas import tpu as pltpu
from jax.experimental.pallas import tpu_sc as plsc

BATCH = 4096
SEQ = 200
EMBED_DIM = 64
N_TOKENS = 20
LANES = 16

NUM_CORES = 2
NUM_SUBCORES = 16
NUM_WORKERS = NUM_CORES * NUM_SUBCORES
ROWS_PER_WORKER = BATCH // NUM_WORKERS

COL0 = 16
PAD_IDS = N_TOKENS - COL0
TOK_W = PAD_IDS + (SEQ - N_TOKENS)
IDS_PER_WORKER = ROWS_PER_WORKER * TOK_W
SPLIT0 = 96
SPLIT1 = TOK_W - SPLIT0
NBUF = 4
LOOKAHEAD = 2


def _body(idx_h, wte_h, learned_h, out_h, tok_v, lv, *bufs):
  stages = bufs[:NBUF]
  gsems = bufs[NBUF:2 * NBUF]
  osems = bufs[2 * NBUF:]

  wid = lax.axis_index("s") * NUM_CORES + lax.axis_index("c")
  base = wid * ROWS_PER_WORKER

  pltpu.sync_copy(idx_h.at[pl.ds(wid * IDS_PER_WORKER, IDS_PER_WORKER)], tok_v)
  pltpu.sync_copy(learned_h.at[pl.ds(COL0, PAD_IDS)], lv)
  patch = [[lv[k, pl.ds(c * LANES, LANES)] for c in range(EMBED_DIM // LANES)]
           for k in range(PAD_IDS)]

  for st in stages:
    pltpu.sync_copy(learned_h.at[pl.ds(0, COL0)], st.at[pl.ds(0, COL0)])

  def issue_gathers(r, st, gsem):
    pltpu.async_copy(
        wte_h.at[tok_v.at[pl.ds(r * TOK_W, SPLIT0)]],
        st.at[pl.ds(COL0, SPLIT0)], gsem)
    pltpu.async_copy(
        wte_h.at[tok_v.at[pl.ds(r * TOK_W + SPLIT0, SPLIT1)]],
        st.at[pl.ds(COL0 + SPLIT0, SPLIT1)], gsem)

  def drain_gathers(st, gsem):
    pltpu.make_async_copy(
        wte_h.at[pl.ds(0, SPLIT0)], st.at[pl.ds(COL0, SPLIT0)], gsem).wait()
    pltpu.make_async_copy(
        wte_h.at[pl.ds(0, SPLIT1)],
        st.at[pl.ds(COL0 + SPLIT0, SPLIT1)], gsem).wait()

  for r0 in range(LOOKAHEAD):
    issue_gathers(r0, stages[r0], gsems[r0])

  def loop_body(i0, carry):
    for p in range(NBUF):
      r = i0 * NBUF + p
      st, gsem, osem = stages[p], gsems[p], osems[p]
      drain_gathers(st, gsem)
      for k in range(PAD_IDS):
        for c in range(EMBED_DIM // LANES):
          st[COL0 + k, pl.ds(c * LANES, LANES)] = patch[k][c]
      pltpu.async_copy(st, out_h.at[base + r], osem)

      pn = (p + LOOKAHEAD) % NBUF
      stn, gsemn, osemn = stages[pn], gsems[pn], osems[pn]

      @pl.when(r + LOOKAHEAD < ROWS_PER_WORKER)
      def _ahead():
        @pl.when(r >= NBUF - LOOKAHEAD)
        def _reclaim():
          pltpu.make_async_copy(stn, out_h.at[base], osemn).wait()
        issue_gathers(r + LOOKAHEAD, stn, gsemn)
    return carry

  lax.fori_loop(0, ROWS_PER_WORKER // NBUF, loop_body, 0)

  for p in range(NBUF):
    pltpu.make_async_copy(stages[p], out_h.at[base], osems[p]).wait()


@jax.jit
def _run(tokens, wte_weight, learned_embedding):
  idx = jnp.pad(tokens[:, N_TOKENS:].astype(jnp.int32),
                ((0, 0), (PAD_IDS, 0))).reshape(-1)

  mesh = plsc.VectorSubcoreMesh(
      core_axis_name="c", subcore_axis_name="s",
      num_cores=NUM_CORES, num_subcores=NUM_SUBCORES)
  return pl.kernel(
      _body,
      out_type=jax.ShapeDtypeStruct((BATCH, SEQ, EMBED_DIM), jnp.float32),
      mesh=mesh,
      compiler_params=pltpu.CompilerParams(use_tc_tiling_on_sc=False),
      scratch_types=(
          [pltpu.VMEM((IDS_PER_WORKER,), jnp.int32),
           pltpu.VMEM((PAD_IDS, EMBED_DIM), jnp.float32)] +
          [pltpu.VMEM((SEQ, EMBED_DIM), jnp.float32)] * NBUF +
          [pltpu.SemaphoreType.DMA] * (2 * NBUF)
      ),
  )(idx, wte_weight, learned_embedding)


def kernel(tokens, wte_weight, learned_embedding):
  return _run(tokens, wte_weight, learned_embedding)


# speedup vs baseline: 1.7269x; 1.7269x over previous
import jax
import jax.numpy as jnp
from jax import lax
from jax.experimental import pallas as pl
from jax.experimental.pallas import tpu as pltpu
from jax.experimental.pallas import tpu_sc as plsc

BATCH = 4096
SEQ = 200
EMBED_DIM = 64
N_TOKENS = 20
LANES = 16

NUM_CORES = 2
NUM_SUBCORES = 16
NUM_WORKERS = NUM_CORES * NUM_SUBCORES
ROWS_PER_WORKER = BATCH // NUM_WORKERS

COL0 = 16
TOK_W = SEQ - COL0
SPLIT0 = 96
SPLIT1 = TOK_W - SPLIT0
NBUF = 4
LOOKAHEAD = 2


def _body(tokens_h, wte_h, learned_h, out_h, tok_v, lv, *bufs):
  stages = bufs[:NBUF]
  gsems = bufs[NBUF:2 * NBUF]
  osems = bufs[2 * NBUF:]

  wid = lax.axis_index("s") * NUM_CORES + lax.axis_index("c")
  base = wid * ROWS_PER_WORKER

  pltpu.sync_copy(
      tokens_h.at[pl.ds(base, ROWS_PER_WORKER), pl.ds(COL0, TOK_W)], tok_v)
  pltpu.sync_copy(learned_h.at[pl.ds(COL0, N_TOKENS - COL0)], lv)
  patch = [[lv[k, pl.ds(c * LANES, LANES)] for c in range(EMBED_DIM // LANES)]
           for k in range(N_TOKENS - COL0)]

  for st in stages:
    pltpu.sync_copy(learned_h.at[pl.ds(0, COL0)], st.at[pl.ds(0, COL0)])

  def issue_gathers(r, st, gsem):
    pltpu.async_copy(
        wte_h.at[tok_v.at[r, pl.ds(0, SPLIT0)]],
        st.at[pl.ds(COL0, SPLIT0)], gsem)
    pltpu.async_copy(
        wte_h.at[tok_v.at[r, pl.ds(SPLIT0, SPLIT1)]],
        st.at[pl.ds(COL0 + SPLIT0, SPLIT1)], gsem)

  def drain_gathers(st, gsem):
    pltpu.make_async_copy(
        wte_h.at[pl.ds(0, SPLIT0)], st.at[pl.ds(COL0, SPLIT0)], gsem).wait()
    pltpu.make_async_copy(
        wte_h.at[pl.ds(0, SPLIT1)],
        st.at[pl.ds(COL0 + SPLIT0, SPLIT1)], gsem).wait()

  for r0 in range(LOOKAHEAD):
    issue_gathers(r0, stages[r0], gsems[r0])

  def loop_body(i0, carry):
    for p in range(NBUF):
      r = i0 * NBUF + p
      st, gsem, osem = stages[p], gsems[p], osems[p]
      drain_gathers(st, gsem)
      for k in range(N_TOKENS - COL0):
        for c in range(EMBED_DIM // LANES):
          st[COL0 + k, pl.ds(c * LANES, LANES)] = patch[k][c]
      pltpu.async_copy(st, out_h.at[base + r], osem)

      pn = (p + LOOKAHEAD) % NBUF
      stn, gsemn, osemn = stages[pn], gsems[pn], osems[pn]

      @pl.when(r + LOOKAHEAD < ROWS_PER_WORKER)
      def _ahead():
        @pl.when(r >= NBUF - LOOKAHEAD)
        def _reclaim():
          pltpu.make_async_copy(stn, out_h.at[base], osemn).wait()
        issue_gathers(r + LOOKAHEAD, stn, gsemn)
    return carry

  lax.fori_loop(0, ROWS_PER_WORKER // NBUF, loop_body, 0)

  for p in range(NBUF):
    pltpu.make_async_copy(stages[p], out_h.at[base], osems[p]).wait()


@jax.jit
def _run(tokens, wte_weight, learned_embedding):
  mesh = plsc.VectorSubcoreMesh(
      core_axis_name="c", subcore_axis_name="s",
      num_cores=NUM_CORES, num_subcores=NUM_SUBCORES)
  return pl.kernel(
      _body,
      out_type=jax.ShapeDtypeStruct((BATCH, SEQ, EMBED_DIM), jnp.float32),
      mesh=mesh,
      compiler_params=pltpu.CompilerParams(use_tc_tiling_on_sc=False),
      scratch_types=(
          [pltpu.VMEM((ROWS_PER_WORKER, TOK_W), jnp.int32),
           pltpu.VMEM((N_TOKENS - COL0, EMBED_DIM), jnp.float32)] +
          [pltpu.VMEM((SEQ, EMBED_DIM), jnp.float32)] * NBUF +
          [pltpu.SemaphoreType.DMA] * (2 * NBUF)
      ),
  )(tokens, wte_weight, learned_embedding)


def kernel(tokens, wte_weight, learned_embedding):
  return _run(tokens.astype(jnp.int32), wte_weight, learned_embedding)
